# Initial kernel scaffold; baseline (speedup 1.0000x reference)
import jax
import jax.numpy as jnp
from jax.experimental import pallas as pl


def kernel(nodeFeatures, edgeIndex, edgeFeatures, W, b, We, be):
    raise NotImplementedError("write your pallas kernel here")


# speedup vs baseline: 2.6878x; 2.6878x over previous
import functools

import jax
import jax.numpy as jnp
from jax import lax
from jax.experimental import pallas as pl
from jax.experimental.pallas import tpu as pltpu
from jax.experimental.pallas import tpu_sc as plsc

N = 10000
E = 320000
D = 128
DE = 6
EP = 16
NC = 2
NS = 16
NW = NC * NS
EPW = E // NW
K = 80
NCH = EPW // K
G = 25
NG = NCH // G
NBLK = N // K
MAXB = 8


def _sc_aggregate(x, src4d, dst1d, efp):
    mesh = plsc.VectorSubcoreMesh(core_axis_name="c", subcore_axis_name="s")

    @functools.partial(
        pl.kernel,
        mesh=mesh,
        out_type=[
            jax.ShapeDtypeStruct((NC, N, D), jnp.float32),
            jax.ShapeDtypeStruct((NC, N, D), jnp.float32),
        ],
        scratch_types=[
            pltpu.VMEM((G, K), jnp.int32),
            pltpu.VMEM((K,), jnp.int32),
            pltpu.VMEM((K, D), jnp.float32),
            pltpu.VMEM((K, EP), jnp.float32),
            pltpu.VMEM_SHARED((N, D), jnp.float32),
            pltpu.SemaphoreType.DMA,
        ],
    )
    def agg(x_hbm, src_hbm, dst_hbm, ef_hbm, outx_hbm, oute_hbm,
            src_v, dstc_v, rows_v, ef_v, acc_s, sem):
        cid = lax.axis_index("c")
        sid = lax.axis_index("s")
        wid = cid * NS + sid
        ebase = wid * EPW

        zeros = jnp.zeros((16,), jnp.float32)

        def zero_rows_buf():
            def zrows_body(i, carry):
                rows_v[i // (D // 16), pl.ds((i % (D // 16)) * 16, 16)] = zeros
                return carry

            lax.fori_loop(0, K * (D // 16), zrows_body, 0)

        def zero_acc_blocks():
            def init_body(m, carry):
                blk = sid + m * NS

                @pl.when(blk < NBLK)
                def _do():
                    pltpu.sync_copy(rows_v, acc_s.at[pl.ds(blk * K, K)])

                return carry

            lax.fori_loop(0, MAXB, init_body, 0)

        def copy_acc_out(out_hbm):
            def out_body(m, carry):
                blk = sid + m * NS

                @pl.when(blk < NBLK)
                def _do():
                    r0 = blk * K
                    pltpu.sync_copy(acc_s.at[pl.ds(r0, K)], rows_v)
                    pltpu.sync_copy(rows_v, out_hbm.at[cid, pl.ds(r0, K)])

                return carry

            lax.fori_loop(0, MAXB, out_body, 0)

        zero_rows_buf()
        zero_acc_blocks()
        plsc.subcore_barrier()

        def group_body(g, carry):
            pltpu.sync_copy(src_hbm.at[wid, g], src_v)

            def chunk_body(c, carry2):
                e0 = ebase + (g * G + c) * K
                pltpu.sync_copy(dst_hbm.at[pl.ds(e0, K)], dstc_v)
                pltpu.async_copy(x_hbm.at[src_v.at[c]], rows_v, sem).wait()
                pltpu.sync_copy(rows_v, acc_s.at[dstc_v], add=True)
                return carry2

            lax.fori_loop(0, G, chunk_body, 0)
            return carry

        lax.fori_loop(0, NG, group_body, 0)

        plsc.subcore_barrier()
        copy_acc_out(outx_hbm)
        plsc.subcore_barrier()

        zero_rows_buf()
        zero_acc_blocks()
        plsc.subcore_barrier()

        def chunkb_body(c, carry):
            e0 = ebase + c * K
            pltpu.sync_copy(dst_hbm.at[pl.ds(e0, K)], dstc_v)
            pltpu.sync_copy(ef_hbm.at[pl.ds(e0, K)], ef_v)

            def expand_body(j, carry2):
                rows_v[j, pl.ds(0, EP)] = ef_v[j, :]
                return carry2

            lax.fori_loop(0, K, expand_body, 0)
            pltpu.sync_copy(rows_v, acc_s.at[dstc_v], add=True)
            return carry

        lax.fori_loop(0, NCH, chunkb_body, 0)

        plsc.subcore_barrier()
        copy_acc_out(oute_hbm)

    return agg(x, src4d, dst1d, efp)


_BLK = 1000


def _tc_body(accx_ref, acce_ref, wt_ref, wep_ref, o_ref):
    a = accx_ref[0] + accx_ref[1]
    e = acce_ref[0] + acce_ref[1]
    y = jax.lax.dot(a, wt_ref[...], precision=jax.lax.Precision.HIGHEST,
                    preferred_element_type=jnp.float32)
    y = y + jax.lax.dot(e, wep_ref[...], precision=jax.lax.Precision.HIGHEST,
                        preferred_element_type=jnp.float32)
    o_ref[...] = jnp.maximum(y, 0.0)


def _tc_combine(accx, acce, wt, wep):
    return pl.pallas_call(
        _tc_body,
        grid=(N // _BLK,),
        in_specs=[
            pl.BlockSpec((NC, _BLK, D), lambda i: (0, i, 0)),
            pl.BlockSpec((NC, _BLK, D), lambda i: (0, i, 0)),
            pl.BlockSpec((D, D), lambda i: (0, 0)),
            pl.BlockSpec((D, D), lambda i: (0, 0)),
        ],
        out_specs=pl.BlockSpec((_BLK, D), lambda i: (i, 0)),
        out_shape=jax.ShapeDtypeStruct((N, D), jnp.float32),
    )(accx, acce, wt, wep)


def kernel(nodeFeatures, edgeIndex, edgeFeatures, W, b, We, be):
    ei = edgeIndex.astype(jnp.int32)
    src4d = ei[:, 0].reshape(NW, NG, G, K)
    dst1d = ei[:, 1]
    efp = jnp.concatenate(
        [
            edgeFeatures.astype(jnp.float32),
            jnp.ones((E, 1), jnp.float32),
            jnp.zeros((E, EP - DE - 1), jnp.float32),
        ],
        axis=1,
    )
    accx, acce = _sc_aggregate(nodeFeatures, src4d, dst1d, efp)

    wt = W.T
    wep = (
        jnp.zeros((D, D), jnp.float32)
        .at[:DE].set(We.T)
        .at[DE].set(b + be)
    )
    return _tc_combine(accx, acce, wt, wep)


# speedup vs baseline: 2.9010x; 1.0793x over previous
import functools

import jax
import jax.numpy as jnp
from jax import lax
from jax.experimental import pallas as pl
from jax.experimental.pallas import tpu as pltpu
from jax.experimental.pallas import tpu_sc as plsc

N = 10000
E = 320000
D = 128
DE = 6
EP = 16
NC = 2
NS = 16
NW = NC * NS
EPW = E // NW
K = 80
NCH = EPW // K
NBLK = N // K
MAXB = 8


def _sc_aggregate(x, src1d, dst1d, efp):
    mesh = plsc.VectorSubcoreMesh(core_axis_name="c", subcore_axis_name="s")

    @functools.partial(
        pl.kernel,
        mesh=mesh,
        out_type=[
            jax.ShapeDtypeStruct((NC, N, D), jnp.float32),
            jax.ShapeDtypeStruct((NC, N, D), jnp.float32),
        ],
        scratch_types=[
            pltpu.VMEM((K,), jnp.int32),
            pltpu.VMEM((K,), jnp.int32),
            pltpu.VMEM((K,), jnp.int32),
            pltpu.VMEM((K,), jnp.int32),
            pltpu.VMEM((K, D), jnp.float32),
            pltpu.VMEM((K, D), jnp.float32),
            pltpu.VMEM((K, EP), jnp.float32),
            pltpu.VMEM((K, EP), jnp.float32),
            pltpu.VMEM_SHARED((N, D), jnp.float32),
            pltpu.SemaphoreType.DMA,
            pltpu.SemaphoreType.DMA,
            pltpu.SemaphoreType.DMA,
            pltpu.SemaphoreType.DMA,
        ],
    )
    def agg(x_hbm, src_hbm, dst_hbm, ef_hbm, outx_hbm, oute_hbm,
            src0_v, src1_v, dst0_v, dst1_v, rows0_v, rows1_v, ef0_v, ef1_v,
            acc_s, semg0, semg1, sems0, sems1):
        cid = lax.axis_index("c")
        sid = lax.axis_index("s")
        wid = cid * NS + sid
        ebase = wid * EPW

        srcb = (src0_v, src1_v)
        dstb = (dst0_v, dst1_v)
        rowsb = (rows0_v, rows1_v)
        efb = (ef0_v, ef1_v)
        semg = (semg0, semg1)
        sems = (sems0, sems1)

        zeros = jnp.zeros((16,), jnp.float32)

        def zero_rows_bufs():
            def zx0(i, carry):
                rows0_v[i // (D // 16), pl.ds((i % (D // 16)) * 16, 16)] = zeros
                return carry

            def zx1(i, carry):
                rows1_v[i // (D // 16), pl.ds((i % (D // 16)) * 16, 16)] = zeros
                return carry

            lax.fori_loop(0, K * (D // 16), zx0, 0)
            lax.fori_loop(0, K * (D // 16), zx1, 0)

        def zero_acc_blocks():
            def init_body(m, carry):
                blk = sid + m * NS

                @pl.when(blk < NBLK)
                def _do():
                    pltpu.sync_copy(rows0_v, acc_s.at[pl.ds(blk * K, K)])

                return carry

            lax.fori_loop(0, MAXB, init_body, 0)

        def copy_acc_out(out_hbm):
            def out_body(m, carry):
                blk = sid + m * NS

                @pl.when(blk < NBLK)
                def _do():
                    r0 = blk * K
                    pltpu.sync_copy(acc_s.at[pl.ds(r0, K)], rows0_v)
                    pltpu.sync_copy(rows0_v, out_hbm.at[cid, pl.ds(r0, K)])

                return carry

            lax.fori_loop(0, MAXB, out_body, 0)

        def wait_scatter(b):
            pltpu.make_async_copy(
                rowsb[b], acc_s.at[dstb[b]], sems[b]).wait()

        zero_rows_bufs()
        zero_acc_blocks()
        plsc.subcore_barrier()

        def chunk_a(c, b, guarded):
            e0 = ebase + c * K
            if guarded:
                wait_scatter(b)
            pltpu.sync_copy(src_hbm.at[pl.ds(e0, K)], srcb[b])
            pltpu.sync_copy(dst_hbm.at[pl.ds(e0, K)], dstb[b])
            pltpu.async_copy(x_hbm.at[srcb[b]], rowsb[b], semg[b]).wait()
            pltpu.async_copy(rowsb[b], acc_s.at[dstb[b]], sems[b], add=True)

        chunk_a(0, 0, False)
        chunk_a(1, 1, False)

        def loopa_body(i, carry):
            c = 2 * i

            @pl.when(i > 0)
            def _do():
                chunk_a(c, 0, True)
                chunk_a(c + 1, 1, True)

            return carry

        lax.fori_loop(1, (NCH - 1) // 2, loopa_body, 0)
        chunk_a(NCH - 1, 0, True)
        wait_scatter(0)
        wait_scatter(1)

        plsc.subcore_barrier()
        copy_acc_out(outx_hbm)
        plsc.subcore_barrier()

        zero_rows_bufs()
        zero_acc_blocks()
        plsc.subcore_barrier()

        def chunk_b(c, b, guarded):
            e0 = ebase + c * K
            if guarded:
                wait_scatter(b)
            pltpu.sync_copy(dst_hbm.at[pl.ds(e0, K)], dstb[b])
            pltpu.sync_copy(ef_hbm.at[pl.ds(e0, K)], efb[b])

            rv, ev = rowsb[b], efb[b]

            def expand_body(j, carry2):
                rv[j, pl.ds(0, EP)] = ev[j, :]
                return carry2

            lax.fori_loop(0, K, expand_body, 0)
            pltpu.async_copy(rowsb[b], acc_s.at[dstb[b]], sems[b], add=True)

        chunk_b(0, 0, False)
        chunk_b(1, 1, False)

        def loopb_body(i, carry):
            c = 2 * i

            @pl.when(i > 0)
            def _do():
                chunk_b(c, 0, True)
                chunk_b(c + 1, 1, True)

            return carry

        lax.fori_loop(1, (NCH - 1) // 2, loopb_body, 0)
        chunk_b(NCH - 1, 0, True)
        wait_scatter(0)
        wait_scatter(1)

        plsc.subcore_barrier()
        copy_acc_out(oute_hbm)

    return agg(x, src1d, dst1d, efp)


_BLK = 1000


def _tc_body(accx_ref, acce_ref, wt_ref, wep_ref, o_ref):
    a = accx_ref[0] + accx_ref[1]
    e = acce_ref[0] + acce_ref[1]
    y = jax.lax.dot(a, wt_ref[...], precision=jax.lax.Precision.HIGHEST,
                    preferred_element_type=jnp.float32)
    y = y + jax.lax.dot(e, wep_ref[...], precision=jax.lax.Precision.HIGHEST,
                        preferred_element_type=jnp.float32)
    o_ref[...] = jnp.maximum(y, 0.0)


def _tc_combine(accx, acce, wt, wep):
    return pl.pallas_call(
        _tc_body,
        grid=(N // _BLK,),
        in_specs=[
            pl.BlockSpec((NC, _BLK, D), lambda i: (0, i, 0)),
            pl.BlockSpec((NC, _BLK, D), lambda i: (0, i, 0)),
            pl.BlockSpec((D, D), lambda i: (0, 0)),
            pl.BlockSpec((D, D), lambda i: (0, 0)),
        ],
        out_specs=pl.BlockSpec((_BLK, D), lambda i: (i, 0)),
        out_shape=jax.ShapeDtypeStruct((N, D), jnp.float32),
    )(accx, acce, wt, wep)


def kernel(nodeFeatures, edgeIndex, edgeFeatures, W, b, We, be):
    ei = edgeIndex.astype(jnp.int32)
    src1d = ei[:, 0]
    dst1d = ei[:, 1]
    efp = jnp.concatenate(
        [
            edgeFeatures.astype(jnp.float32),
            jnp.ones((E, 1), jnp.float32),
            jnp.zeros((E, EP - DE - 1), jnp.float32),
        ],
        axis=1,
    )
    accx, acce = _sc_aggregate(nodeFeatures, src1d, dst1d, efp)

    wt = W.T
    wep = (
        jnp.zeros((D, D), jnp.float32)
        .at[:DE].set(We.T)
        .at[DE].set(b + be)
    )
    return _tc_combine(accx, acce, wt, wep)


# speedup vs baseline: 4.0720x; 1.4036x over previous
import functools

import jax
import jax.numpy as jnp
from jax import lax
from jax.experimental import pallas as pl
from jax.experimental.pallas import tpu as pltpu
from jax.experimental.pallas import tpu_sc as plsc

N = 10000
E = 320000
D = 128
DE = 6
EP = 16
EPR = D // EP
NC = 2
NS = 16
NW = NC * NS
EPW = E // NW
K = 80
NCH = EPW // K
GB = 5
NB = NCH // GB
NBLK = N // K
MAXB = 8


def _sc_aggregate(x, src5d, dst5d, efp):
    mesh = plsc.VectorSubcoreMesh(core_axis_name="c", subcore_axis_name="s")

    @functools.partial(
        pl.kernel,
        mesh=mesh,
        out_type=[
            jax.ShapeDtypeStruct((NC, N, D), jnp.float32),
            jax.ShapeDtypeStruct((NC, N, D), jnp.float32),
        ],
        scratch_types=[
            pltpu.VMEM((GB, K), jnp.int32),
            pltpu.VMEM((GB, K), jnp.int32),
            pltpu.VMEM((K,), jnp.int32),
            pltpu.VMEM((K,), jnp.int32),
            pltpu.VMEM((K, D), jnp.float32),
            pltpu.VMEM((K, D), jnp.float32),
            pltpu.VMEM((GB * K // EPR, D), jnp.float32),
            pltpu.VMEM_SHARED((N, D), jnp.float32),
            pltpu.SemaphoreType.DMA,
            pltpu.SemaphoreType.DMA,
            pltpu.SemaphoreType.DMA,
            pltpu.SemaphoreType.DMA,
        ],
    )
    def agg(x_hbm, src_hbm, dst_hbm, ef_hbm, outx_hbm, oute_hbm,
            src_v, dst_v, dstc0_v, dstc1_v, rows0_v, rows1_v, ef_v,
            acc_s, semg0, semg1, sems0, sems1):
        cid = lax.axis_index("c")
        sid = lax.axis_index("s")
        wid = cid * NS + sid

        rowsb = (rows0_v, rows1_v)
        dstcb = (dstc0_v, dstc1_v)
        semg = (semg0, semg1)
        sems = (sems0, sems1)

        zeros = jnp.zeros((16,), jnp.float32)

        def stage_dst(ci, b):
            for j in range(K // 16):
                dstcb[b][pl.ds(j * 16, 16)] = dst_v[ci, pl.ds(j * 16, 16)]

        def zero_rows_bufs():
            def zx0(i, carry):
                rows0_v[i // (D // 16), pl.ds((i % (D // 16)) * 16, 16)] = zeros
                return carry

            def zx1(i, carry):
                rows1_v[i // (D // 16), pl.ds((i % (D // 16)) * 16, 16)] = zeros
                return carry

            lax.fori_loop(0, K * (D // 16), zx0, 0)
            lax.fori_loop(0, K * (D // 16), zx1, 0)

        def zero_acc_blocks():
            def init_body(m, carry):
                blk = sid + m * NS

                @pl.when(blk < NBLK)
                def _do():
                    pltpu.sync_copy(rows0_v, acc_s.at[pl.ds(blk * K, K)])

                return carry

            lax.fori_loop(0, MAXB, init_body, 0)

        def copy_acc_out(out_hbm):
            def out_body(m, carry):
                blk = sid + m * NS

                @pl.when(blk < NBLK)
                def _do():
                    r0 = blk * K
                    pltpu.sync_copy(acc_s.at[pl.ds(r0, K)], rows0_v)
                    pltpu.sync_copy(rows0_v, out_hbm.at[cid, pl.ds(r0, K)])

                return carry

            lax.fori_loop(0, MAXB, out_body, 0)

        def wait_scatter(b):
            pltpu.make_async_copy(
                rowsb[b], acc_s.at[dstcb[b]], sems[b]).wait()

        zero_rows_bufs()
        zero_acc_blocks()
        plsc.subcore_barrier()

        def batch_a(bt, first):
            pltpu.sync_copy(src_hbm.at[wid, bt], src_v)
            pltpu.sync_copy(dst_hbm.at[wid, bt], dst_v)
            for ci in range(GB):
                b = ci % 2
                if not (first and ci < 2):
                    wait_scatter(b)
                stage_dst(ci, b)
                pltpu.async_copy(
                    x_hbm.at[src_v.at[ci]], rowsb[b], semg[b]).wait()
                pltpu.async_copy(
                    rowsb[b], acc_s.at[dstcb[b]], sems[b], add=True)

        batch_a(0, True)

        def loopa_body(bt, carry):
            batch_a(bt, False)
            return carry

        lax.fori_loop(1, NB, loopa_body, 0)
        wait_scatter(0)
        wait_scatter(1)

        plsc.subcore_barrier()
        copy_acc_out(outx_hbm)
        plsc.subcore_barrier()

        zero_rows_bufs()
        zero_acc_blocks()
        plsc.subcore_barrier()

        def batch_b(bt, first):
            pltpu.sync_copy(dst_hbm.at[wid, bt], dst_v)
            pltpu.sync_copy(ef_hbm.at[wid, bt], ef_v)
            for ci in range(GB):
                b = ci % 2
                if not (first and ci < 2):
                    wait_scatter(b)
                stage_dst(ci, b)

                rv = rowsb[b]
                pr0 = ci * (K // EPR)

                def expand_body(pr, carry2):
                    for jj in range(EPR):
                        rv[pr * EPR + jj, pl.ds(0, EP)] = (
                            ef_v[pr0 + pr, pl.ds(jj * EP, EP)])
                    return carry2

                lax.fori_loop(0, K // EPR, expand_body, 0)
                pltpu.async_copy(
                    rowsb[b], acc_s.at[dstcb[b]], sems[b], add=True)

        batch_b(0, True)

        def loopb_body(bt, carry):
            batch_b(bt, False)
            return carry

        lax.fori_loop(1, NB, loopb_body, 0)
        wait_scatter(0)
        wait_scatter(1)

        plsc.subcore_barrier()
        copy_acc_out(oute_hbm)

    return agg(x, src5d, dst5d, efp)


_BLK = 1000


def _tc_body(accx_ref, acce_ref, wt_ref, wep_ref, o_ref):
    a = accx_ref[0] + accx_ref[1]
    e = acce_ref[0] + acce_ref[1]
    y = jax.lax.dot(a, wt_ref[...], precision=jax.lax.Precision.HIGHEST,
                    preferred_element_type=jnp.float32)
    y = y + jax.lax.dot(e, wep_ref[...], precision=jax.lax.Precision.HIGHEST,
                        preferred_element_type=jnp.float32)
    o_ref[...] = jnp.maximum(y, 0.0)


def _tc_combine(accx, acce, wt, wep):
    return pl.pallas_call(
        _tc_body,
        grid=(N // _BLK,),
        in_specs=[
            pl.BlockSpec((NC, _BLK, D), lambda i: (0, i, 0)),
            pl.BlockSpec((NC, _BLK, D), lambda i: (0, i, 0)),
            pl.BlockSpec((D, D), lambda i: (0, 0)),
            pl.BlockSpec((D, D), lambda i: (0, 0)),
        ],
        out_specs=pl.BlockSpec((_BLK, D), lambda i: (i, 0)),
        out_shape=jax.ShapeDtypeStruct((N, D), jnp.float32),
    )(accx, acce, wt, wep)


def kernel(nodeFeatures, edgeIndex, edgeFeatures, W, b, We, be):
    ei = edgeIndex.astype(jnp.int32)
    src5d = ei[:, 0].reshape(NW, NB, GB, K)
    dst5d = ei[:, 1].reshape(NW, NB, GB, K)
    efp = jnp.concatenate(
        [
            edgeFeatures.astype(jnp.float32),
            jnp.ones((E, 1), jnp.float32),
            jnp.zeros((E, EP - DE - 1), jnp.float32),
        ],
        axis=1,
    )
    efp = efp.reshape(NW, NB, GB * K // EPR, D)
    accx, acce = _sc_aggregate(nodeFeatures, src5d, dst5d, efp)

    wt = W.T
    wep = (
        jnp.zeros((D, D), jnp.float32)
        .at[:DE].set(We.T)
        .at[DE].set(b + be)
    )
    return _tc_combine(accx, acce, wt, wep)


# speedup vs baseline: 4.3438x; 1.0668x over previous
import functools

import jax
import jax.numpy as jnp
from jax import lax
from jax.experimental import pallas as pl
from jax.experimental.pallas import tpu as pltpu
from jax.experimental.pallas import tpu_sc as plsc

N = 10000
E = 320000
D = 128
DE = 6
EP = 16
EPR = D // EP
NC = 2
NS = 16
NW = NC * NS
EPW = E // NW
K = 80
NCH = EPW // K
GB = 5
NB = NCH // GB
NBLK = N // K
MAXB = 8


def _sc_aggregate(x, src5d, dst5d, efp):
    mesh = plsc.VectorSubcoreMesh(core_axis_name="c", subcore_axis_name="s")

    @functools.partial(
        pl.kernel,
        mesh=mesh,
        out_type=[
            jax.ShapeDtypeStruct((NC, N, D), jnp.float32),
            jax.ShapeDtypeStruct((NC, N, D), jnp.float32),
        ],
        scratch_types=[
            pltpu.VMEM((GB, K), jnp.int32),
            pltpu.VMEM((GB, K), jnp.int32),
            pltpu.VMEM((K,), jnp.int32),
            pltpu.VMEM((K,), jnp.int32),
            pltpu.VMEM((K, D), jnp.float32),
            pltpu.VMEM((K, D), jnp.float32),
            pltpu.VMEM((GB * K // EPR, D), jnp.float32),
            pltpu.VMEM_SHARED((N, D), jnp.float32),
            pltpu.SemaphoreType.DMA,
            pltpu.SemaphoreType.DMA,
            pltpu.SemaphoreType.DMA,
            pltpu.SemaphoreType.DMA,
            pltpu.SemaphoreType.DMA,
            pltpu.SemaphoreType.DMA,
        ],
    )
    def agg(x_hbm, src_hbm, dst_hbm, ef_hbm, outx_hbm, oute_hbm,
            src_v, dst_v, dstc0_v, dstc1_v, rows0_v, rows1_v, ef_v,
            acc_s, semg0, semg1, sems0, sems1, semi0, semi1):
        cid = lax.axis_index("c")
        sid = lax.axis_index("s")
        wid = cid * NS + sid

        rowsb = (rows0_v, rows1_v)
        dstcb = (dstc0_v, dstc1_v)
        semg = (semg0, semg1)
        sems = (sems0, sems1)

        zeros = jnp.zeros((16,), jnp.float32)

        def stage_dst(ci, b):
            for j in range(K // 16):
                dstcb[b][pl.ds(j * 16, 16)] = dst_v[ci, pl.ds(j * 16, 16)]

        def zero_rows_bufs():
            def zx0(i, carry):
                rows0_v[i // (D // 16), pl.ds((i % (D // 16)) * 16, 16)] = zeros
                return carry

            def zx1(i, carry):
                rows1_v[i // (D // 16), pl.ds((i % (D // 16)) * 16, 16)] = zeros
                return carry

            lax.fori_loop(0, K * (D // 16), zx0, 0)
            lax.fori_loop(0, K * (D // 16), zx1, 0)

        def zero_acc_blocks():
            def init_body(m, carry):
                blk = sid + m * NS

                @pl.when(blk < NBLK)
                def _do():
                    pltpu.sync_copy(rows0_v, acc_s.at[pl.ds(blk * K, K)])

                return carry

            lax.fori_loop(0, MAXB, init_body, 0)

        semib = (semi0, semi1)

        def copy_acc_out(out_hbm):
            for m in range(MAXB):
                blk = sid + m * NS
                b = m % 2

                @pl.when(blk < NBLK)
                def _do(blk=blk, b=b, m=m):
                    r0 = blk * K
                    if m >= 2:
                        pltpu.make_async_copy(
                            rowsb[b], out_hbm.at[cid, pl.ds(r0, K)],
                            semib[b]).wait()
                    pltpu.async_copy(
                        acc_s.at[pl.ds(r0, K)], rowsb[b], semg[b]).wait()
                    pltpu.async_copy(
                        rowsb[b], out_hbm.at[cid, pl.ds(r0, K)], semib[b])

            for b in range(2):
                pltpu.make_async_copy(
                    rowsb[b], out_hbm.at[cid, pl.ds(0, K)], semib[b]).wait()

        def wait_scatter(b):
            pltpu.make_async_copy(
                rowsb[b], acc_s.at[dstcb[b]], sems[b]).wait()

        zero_rows_bufs()
        zero_acc_blocks()
        plsc.subcore_barrier()

        def batch_a(bt, first):
            d1 = pltpu.async_copy(src_hbm.at[wid, bt], src_v, semi0)
            d2 = pltpu.async_copy(dst_hbm.at[wid, bt], dst_v, semi1)
            d1.wait()
            d2.wait()
            for ci in range(GB):
                b = ci % 2
                if not (first and ci < 2):
                    wait_scatter(b)
                stage_dst(ci, b)
                pltpu.async_copy(
                    x_hbm.at[src_v.at[ci]], rowsb[b], semg[b]).wait()
                pltpu.async_copy(
                    rowsb[b], acc_s.at[dstcb[b]], sems[b], add=True)

        batch_a(0, True)

        def loopa_body(bt, carry):
            batch_a(bt, False)
            return carry

        lax.fori_loop(1, NB, loopa_body, 0)
        wait_scatter(0)
        wait_scatter(1)

        plsc.subcore_barrier()
        copy_acc_out(outx_hbm)
        plsc.subcore_barrier()

        zero_rows_bufs()
        zero_acc_blocks()
        plsc.subcore_barrier()

        def batch_b(bt, first):
            d1 = pltpu.async_copy(dst_hbm.at[wid, bt], dst_v, semi0)
            d2 = pltpu.async_copy(ef_hbm.at[wid, bt], ef_v, semi1)
            d1.wait()
            d2.wait()
            for ci in range(GB):
                b = ci % 2
                if not (first and ci < 2):
                    wait_scatter(b)
                stage_dst(ci, b)

                rv = rowsb[b]
                pr0 = ci * (K // EPR)

                def expand_body(pr, carry2):
                    for jj in range(EPR):
                        rv[pr * EPR + jj, pl.ds(0, EP)] = (
                            ef_v[pr0 + pr, pl.ds(jj * EP, EP)])
                    return carry2

                lax.fori_loop(0, K // EPR, expand_body, 0)
                pltpu.async_copy(
                    rowsb[b], acc_s.at[dstcb[b]], sems[b], add=True)

        batch_b(0, True)

        def loopb_body(bt, carry):
            batch_b(bt, False)
            return carry

        lax.fori_loop(1, NB, loopb_body, 0)
        wait_scatter(0)
        wait_scatter(1)

        plsc.subcore_barrier()
        copy_acc_out(oute_hbm)

    return agg(x, src5d, dst5d, efp)


_BLK = 1000


def _tc_body(accx_ref, acce_ref, wt_ref, wep_ref, o_ref):
    a = accx_ref[0] + accx_ref[1]
    e = acce_ref[0] + acce_ref[1]
    y = jax.lax.dot(a, wt_ref[...], precision=jax.lax.Precision.HIGHEST,
                    preferred_element_type=jnp.float32)
    y = y + jax.lax.dot(e, wep_ref[...], precision=jax.lax.Precision.HIGHEST,
                        preferred_element_type=jnp.float32)
    o_ref[...] = jnp.maximum(y, 0.0)


def _tc_combine(accx, acce, wt, wep):
    return pl.pallas_call(
        _tc_body,
        grid=(N // _BLK,),
        in_specs=[
            pl.BlockSpec((NC, _BLK, D), lambda i: (0, i, 0)),
            pl.BlockSpec((NC, _BLK, D), lambda i: (0, i, 0)),
            pl.BlockSpec((D, D), lambda i: (0, 0)),
            pl.BlockSpec((D, D), lambda i: (0, 0)),
        ],
        out_specs=pl.BlockSpec((_BLK, D), lambda i: (i, 0)),
        out_shape=jax.ShapeDtypeStruct((N, D), jnp.float32),
    )(accx, acce, wt, wep)


def kernel(nodeFeatures, edgeIndex, edgeFeatures, W, b, We, be):
    ei = edgeIndex.astype(jnp.int32)
    src5d = ei[:, 0].reshape(NW, NB, GB, K)
    dst5d = ei[:, 1].reshape(NW, NB, GB, K)
    efp = jnp.concatenate(
        [
            edgeFeatures.astype(jnp.float32),
            jnp.ones((E, 1), jnp.float32),
            jnp.zeros((E, EP - DE - 1), jnp.float32),
        ],
        axis=1,
    )
    efp = efp.reshape(NW, NB, GB * K // EPR, D)
    accx, acce = _sc_aggregate(nodeFeatures, src5d, dst5d, efp)

    wt = W.T
    wep = (
        jnp.zeros((D, D), jnp.float32)
        .at[:DE].set(We.T)
        .at[DE].set(b + be)
    )
    return _tc_combine(accx, acce, wt, wep)
